# baseline (speedup 1.0000x reference)
import functools

import jax
import jax.numpy as jnp
from jax import lax
from jax.experimental import pallas as pl
from jax.experimental.pallas import tpu as pltpu
from jax.experimental.pallas import tpu_sc as plsc

_DIM = 256
_K = 1024
_TB = 1024


def _argmin_body(x_ref, et2_ref, xx_ref, n_ref, idx_ref):
    m2 = jnp.dot(x_ref[...], et2_ref[...], preferred_element_type=jnp.float32)
    t = (xx_ref[...] - m2) + n_ref[...]
    idx_ref[0, 0, :] = jnp.argmin(t, axis=-1).astype(jnp.int32)


def _nearest_index(flat, et2, xx, en):
    n = flat.shape[0]
    grid = n // _TB
    idx3 = pl.pallas_call(
        _argmin_body,
        grid=(grid,),
        in_specs=[
            pl.BlockSpec((_TB, _DIM), lambda i: (i, 0)),
            pl.BlockSpec((_DIM, _K), lambda i: (0, 0)),
            pl.BlockSpec((_TB, 1), lambda i: (i, 0)),
            pl.BlockSpec((1, _K), lambda i: (0, 0)),
        ],
        out_specs=pl.BlockSpec((1, 1, _TB), lambda i: (i, 0, 0)),
        out_shape=jax.ShapeDtypeStruct((grid, 1, _TB), jnp.int32),
    )(flat, et2, xx, en)
    return idx3.reshape(-1)


@functools.lru_cache(maxsize=None)
def _make_gather(v, d, b):
    info = plsc.get_sparse_core_info()
    nw = info.num_cores * info.num_subcores
    b_per_w = b // nw
    ch = min(b_per_w, 256)
    n_ch = b_per_w // ch
    mesh = plsc.VectorSubcoreMesh(core_axis_name="c", subcore_axis_name="s")

    @functools.partial(
        pl.kernel,
        mesh=mesh,
        out_type=jax.ShapeDtypeStruct((b, d), jnp.float32),
        scratch_types=[
            pltpu.VMEM((ch,), jnp.int32),
            pltpu.VMEM((ch, d), jnp.float32),
            pltpu.SemaphoreType.DMA,
        ],
    )
    def gather(table_hbm, idx_hbm, out_hbm, idx_v, rows_v, sem):
        wid = lax.axis_index("s") * info.num_cores + lax.axis_index("c")
        base = wid * b_per_w
        for c in range(n_ch):
            off = base + c * ch
            pltpu.sync_copy(idx_hbm.at[pl.ds(off, ch)], idx_v)
            pltpu.async_copy(table_hbm.at[idx_v], rows_v, sem).wait()
            pltpu.sync_copy(rows_v, out_hbm.at[pl.ds(off, ch)])

    return gather


def kernel(x, embed):
    b, tok, d = x.shape
    n = b * tok
    flat = x.reshape(-1, d)
    embed_t = embed.T
    xx = jnp.sum(flat**2, axis=1, keepdims=True)
    en = jnp.sum(embed_t**2, axis=0, keepdims=True)
    nc = 2
    cs = n // nc
    et2 = 2.0 * embed_t
    gather = _make_gather(embed.shape[0], d, cs)
    qs, ids = [], []
    for c in range(nc):
        fc = lax.slice_in_dim(flat, c * cs, (c + 1) * cs)
        xc = lax.slice_in_dim(xx, c * cs, (c + 1) * cs)
        ic = _nearest_index(fc, et2, xc, en)
        qs.append(gather(embed, ic))
        ids.append(ic)
    quant = jnp.concatenate(qs)
    idx = jnp.concatenate(ids)
    return quant.reshape(b, tok, d), idx.reshape(b, tok)


# speedup vs baseline: 1.4371x; 1.4371x over previous
import functools

import jax
import jax.numpy as jnp
from jax import lax
from jax.experimental import pallas as pl
from jax.experimental.pallas import tpu as pltpu
from jax.experimental.pallas import tpu_sc as plsc

_DIM = 256
_K = 1024
_TB = 1024


def _argmin_body(x_ref, e_ref, xx_ref, n_ref, idx_ref):
    m2 = lax.dot_general(
        x_ref[...], e_ref[...], (((1,), (1,)), ((), ())),
        preferred_element_type=jnp.float32)
    t = (xx_ref[...] - m2) + n_ref[...]
    idx_ref[0, 0, :] = jnp.argmin(t, axis=-1).astype(jnp.int32)


def _nearest_index(flat, embed, xx, en):
    n = flat.shape[0]
    grid = n // _TB
    idx3 = pl.pallas_call(
        _argmin_body,
        grid=(grid,),
        in_specs=[
            pl.BlockSpec((_TB, _DIM), lambda i: (i, 0)),
            pl.BlockSpec((_K, _DIM), lambda i: (0, 0)),
            pl.BlockSpec((_TB, 1), lambda i: (i, 0)),
            pl.BlockSpec((1, _K), lambda i: (0, 0)),
        ],
        out_specs=pl.BlockSpec((1, 1, _TB), lambda i: (i, 0, 0)),
        out_shape=jax.ShapeDtypeStruct((grid, 1, _TB), jnp.int32),
    )(flat, embed, xx, en)
    return idx3.reshape(-1)


@functools.lru_cache(maxsize=None)
def _make_gather(v, d, b):
    info = plsc.get_sparse_core_info()
    nw = info.num_cores * info.num_subcores
    b_per_w = b // nw
    ch = min(b_per_w, 256)
    n_ch = b_per_w // ch
    mesh = plsc.VectorSubcoreMesh(core_axis_name="c", subcore_axis_name="s")

    @functools.partial(
        pl.kernel,
        mesh=mesh,
        out_type=jax.ShapeDtypeStruct((b, d), jnp.float32),
        scratch_types=[
            pltpu.VMEM((ch,), jnp.int32),
            pltpu.VMEM((ch, d), jnp.float32),
            pltpu.SemaphoreType.DMA,
        ],
    )
    def gather(table_hbm, idx_hbm, out_hbm, idx_v, rows_v, sem):
        wid = lax.axis_index("s") * info.num_cores + lax.axis_index("c")
        base = wid * b_per_w
        for c in range(n_ch):
            off = base + c * ch
            pltpu.sync_copy(idx_hbm.at[pl.ds(off, ch)], idx_v)
            pltpu.async_copy(table_hbm.at[idx_v], rows_v, sem).wait()
            pltpu.sync_copy(rows_v, out_hbm.at[pl.ds(off, ch)])

    return gather


def kernel(x, embed):
    b, tok, d = x.shape
    n = b * tok
    flat = x.reshape(-1, d)
    embed_t = embed.T
    xx = jnp.sum(flat**2, axis=1, keepdims=True)
    en = jnp.sum(embed_t**2, axis=0, keepdims=True)
    idx = _nearest_index(flat, 2.0 * embed, xx, en)
    quant = _make_gather(embed.shape[0], d, n)(embed, idx)
    return quant.reshape(b, tok, d), idx.reshape(b, tok)
